# baseline (speedup 1.0000x reference)
import functools

import jax
import jax.numpy as jnp
from jax import lax
from jax.experimental import pallas as pl
from jax.experimental.pallas import tpu as pltpu
from jax.experimental.pallas import tpu_sc as plsc

NC = 2
NS = 16
NW = NC * NS

CH = 128
BG = 4


def _sc_mesh():
    return plsc.VectorSubcoreMesh(
        core_axis_name="c", subcore_axis_name="s", num_cores=NC, num_subcores=NS
    )


_SC_PARAMS = pltpu.CompilerParams(use_tc_tiling_on_sc=False)


def _zero_fill(ref2d, nrows, width):
    zero = jnp.zeros((16,), jnp.float32)

    def body(j, carry):
        for kk in range(width // 16):
            ref2d[j, pl.ds(kk * 16, 16)] = zero
        return carry

    lax.fori_loop(0, nrows, body, 0)


def _stage_edges(src_hbm, dst_hbm, pad_src_hbm, pad_dst_hbm, src_v, dst_v,
                 base, n_real, n_pad):
    pltpu.sync_copy(src_hbm.at[pl.ds(base, n_real)], src_v.at[pl.ds(0, n_real)])
    pltpu.sync_copy(dst_hbm.at[pl.ds(base, n_real)], dst_v.at[pl.ds(0, n_real)])
    pltpu.sync_copy(pad_src_hbm.at[pl.ds(0, n_pad)],
                    src_v.at[pl.ds(n_real, n_pad)])
    pltpu.sync_copy(pad_dst_hbm.at[pl.ds(0, n_pad)],
                    dst_v.at[pl.ds(n_real, n_pad)])


def _make_deg_kernel(n_acc, ept, cpt, dw):
    rows_per_tile = n_acc // NS
    n_pad = cpt * CH - ept

    @functools.partial(
        pl.kernel,
        out_type=jax.ShapeDtypeStruct((NC, n_acc, dw), jnp.float32),
        mesh=_sc_mesh(),
        compiler_params=_SC_PARAMS,
        scratch_types=[
            pltpu.VMEM((cpt * CH,), jnp.int32),
            pltpu.VMEM((CH, dw), jnp.float32),
            pltpu.VMEM_SHARED((n_acc, dw), jnp.float32),
            pltpu.SemaphoreType.DMA,
        ],
    )
    def k(dst_hbm, pad_dst_hbm, ones_hbm, zin_hbm, out_hbm, dst_v, ones_v,
          acc_sh, sem):
        c = lax.axis_index("c")
        s = lax.axis_index("s")
        w = c * NS + s
        pltpu.sync_copy(dst_hbm.at[pl.ds(w * ept, ept)],
                        dst_v.at[pl.ds(0, ept)])
        pltpu.sync_copy(pad_dst_hbm.at[pl.ds(0, n_pad)],
                        dst_v.at[pl.ds(ept, n_pad)])
        pltpu.sync_copy(ones_hbm, ones_v)
        pltpu.sync_copy(
            zin_hbm.at[pl.ds(s * rows_per_tile, rows_per_tile)],
            acc_sh.at[pl.ds(s * rows_per_tile, rows_per_tile)],
        )
        plsc.subcore_barrier()

        def fire(j, carry):
            pltpu.async_copy(
                ones_v, acc_sh.at[dst_v.at[pl.ds(j * CH, CH)]], sem, add=True
            )
            return carry

        lax.fori_loop(0, cpt, fire, 0)

        def drain(j, carry):
            pltpu.make_async_copy(
                ones_v, acc_sh.at[dst_v.at[pl.ds(j * CH, CH)]], sem
            ).wait()
            return carry

        lax.fori_loop(0, cpt, drain, 0)
        plsc.subcore_barrier()
        pltpu.sync_copy(
            acc_sh.at[pl.ds(s * rows_per_tile, rows_per_tile)],
            out_hbm.at[c, pl.ds(s * rows_per_tile, rows_per_tile)],
        )

    return k


def _pipelined_edge_loop(y_sh, acc_sh, src_v, dst_v, rows_v, semg, sems,
                         cpt, width):
    nb = cpt // BG
    nbp = nb // 2
    assert cpt % (2 * BG) == 0

    def gslice(h, i):
        return rows_v.at[h, pl.ds(i * CH, CH)]

    def eslice(v, b, i):
        return v.at[pl.ds((b * BG + i) * CH, CH)]

    def issue_gathers(b, h):
        for i in range(BG):
            pltpu.async_copy(y_sh.at[eslice(src_v, b, i)], gslice(h, i),
                             semg[h])

    def wait_gathers(b, h):
        for i in range(BG):
            pltpu.make_async_copy(y_sh.at[eslice(src_v, b, i)], gslice(h, i),
                                  semg[h]).wait()

    def issue_scatters(b, h):
        for i in range(BG):
            pltpu.async_copy(gslice(h, i), acc_sh.at[eslice(dst_v, b, i)],
                             sems[h], add=True)

    def wait_scatters(b, h):
        for i in range(BG):
            pltpu.make_async_copy(gslice(h, i), acc_sh.at[eslice(dst_v, b, i)],
                                  sems[h]).wait()

    issue_gathers(0, 0)

    def body(bp, carry):
        b0 = 2 * bp
        wait_gathers(b0, 0)
        issue_scatters(b0, 0)

        @pl.when(bp > 0)
        def _():
            wait_scatters(b0 - 1, 1)

        issue_gathers(b0 + 1, 1)
        wait_gathers(b0 + 1, 1)
        issue_scatters(b0 + 1, 1)

        @pl.when(bp < nbp - 1)
        def _():
            wait_scatters(b0, 0)
            issue_gathers(b0 + 2, 0)

        return carry

    lax.fori_loop(0, nbp, body, 0)
    wait_scatters(nb - 2, 0)
    wait_scatters(nb - 1, 1)


def _make_colsplit_kernel(n_acc, ept, cpt, width):
    rows_per_tile = n_acc // NS
    colw = width // 2
    n_pad = cpt * CH - ept

    @functools.partial(
        pl.kernel,
        out_type=jax.ShapeDtypeStruct((NC, n_acc, colw), jnp.float32),
        mesh=_sc_mesh(),
        compiler_params=_SC_PARAMS,
        scratch_types=[
            pltpu.VMEM((cpt * CH,), jnp.int32),
            pltpu.VMEM((cpt * CH,), jnp.int32),
            pltpu.VMEM((2, BG * CH, colw), jnp.float32),
            pltpu.VMEM_SHARED((n_acc, colw), jnp.float32),
            pltpu.VMEM_SHARED((n_acc, colw), jnp.float32),
            pltpu.SemaphoreType.DMA,
            pltpu.SemaphoreType.DMA,
            pltpu.SemaphoreType.DMA,
            pltpu.SemaphoreType.DMA,
        ],
    )
    def k(y_hbm, src_hbm, dst_hbm, pad_src_hbm, pad_dst_hbm, zin_hbm, out_hbm,
          src_v, dst_v, rows_v, acc_sh, y_sh, semg0, semg1, sems0, sems1):
        c = lax.axis_index("c")
        s = lax.axis_index("s")
        _stage_edges(src_hbm, dst_hbm, pad_src_hbm, pad_dst_hbm,
                     src_v, dst_v, s * ept, ept, n_pad)
        pltpu.sync_copy(
            y_hbm.at[c, pl.ds(s * rows_per_tile, rows_per_tile)],
            y_sh.at[pl.ds(s * rows_per_tile, rows_per_tile)],
        )
        pltpu.sync_copy(
            zin_hbm.at[pl.ds(s * rows_per_tile, rows_per_tile)],
            acc_sh.at[pl.ds(s * rows_per_tile, rows_per_tile)],
        )
        plsc.subcore_barrier()
        _pipelined_edge_loop(y_sh, acc_sh, src_v, dst_v, rows_v,
                             (semg0, semg1), (sems0, sems1), cpt, colw)
        plsc.subcore_barrier()
        pltpu.sync_copy(
            acc_sh.at[pl.ds(s * rows_per_tile, rows_per_tile)],
            out_hbm.at[c, pl.ds(s * rows_per_tile, rows_per_tile)],
        )

    return k


def _make_scatter_kernel(n_acc, ept, cpt, width):
    rows_per_tile = n_acc // NS
    n_pad = cpt * CH - ept

    @functools.partial(
        pl.kernel,
        out_type=jax.ShapeDtypeStruct((NC, n_acc, width), jnp.float32),
        mesh=_sc_mesh(),
        compiler_params=_SC_PARAMS,
        scratch_types=[
            pltpu.VMEM((cpt * CH,), jnp.int32),
            pltpu.VMEM((cpt * CH,), jnp.int32),
            pltpu.VMEM((2, BG * CH, width), jnp.float32),
            pltpu.VMEM_SHARED((n_acc, width), jnp.float32),
            pltpu.VMEM_SHARED((n_acc, width), jnp.float32),
            pltpu.SemaphoreType.DMA,
            pltpu.SemaphoreType.DMA,
            pltpu.SemaphoreType.DMA,
            pltpu.SemaphoreType.DMA,
        ],
    )
    def k(y_hbm, src_hbm, dst_hbm, pad_src_hbm, pad_dst_hbm, out_hbm,
          src_v, dst_v, rows_v, acc_sh, y_sh, semg0, semg1, sems0, sems1):
        c = lax.axis_index("c")
        s = lax.axis_index("s")
        w = c * NS + s
        _stage_edges(src_hbm, dst_hbm, pad_src_hbm, pad_dst_hbm,
                     src_v, dst_v, w * ept, ept, n_pad)
        pltpu.sync_copy(
            y_hbm.at[pl.ds(s * rows_per_tile, rows_per_tile)],
            y_sh.at[pl.ds(s * rows_per_tile, rows_per_tile)],
        )
        _zero_fill(rows_v.at[0], rows_per_tile, width)
        pltpu.sync_copy(rows_v.at[0, pl.ds(0, rows_per_tile)],
                        acc_sh.at[pl.ds(s * rows_per_tile, rows_per_tile)])
        plsc.subcore_barrier()
        _pipelined_edge_loop(y_sh, acc_sh, src_v, dst_v, rows_v,
                             (semg0, semg1), (sems0, sems1), cpt, width)
        plsc.subcore_barrier()
        pltpu.sync_copy(
            acc_sh.at[pl.ds(s * rows_per_tile, rows_per_tile)],
            out_hbm.at[c, pl.ds(s * rows_per_tile, rows_per_tile)],
        )

    return k


def _tc1a_body(x_ref, w1_ref, xw_ref):
    xw_ref[...] = jnp.dot(
        x_ref[...], w1_ref[...], preferred_element_type=jnp.float32
    )


def _tc1b_body(degacc_ref, xw_ref, dinv_ref, y1s_ref):
    d = degacc_ref[...]
    deg = d[0, :, 0:1] + d[1, :, 0:1] + 1.0
    dinv = lax.rsqrt(deg)
    y1 = xw_ref[...] * dinv
    half = y1.shape[1] // 2
    y1s_ref[0] = y1[:, :half]
    y1s_ref[1] = y1[:, half:]
    dinv_ref[...] = jnp.broadcast_to(dinv, dinv_ref.shape)


def _tc2_body(s1s_ref, y1s_ref, dinv_ref, w2_ref, b1_ref, y2s_ref):
    dinv = dinv_ref[...][:, 0:1]
    t = s1s_ref[...] + y1s_ref[...]
    full = jnp.concatenate([t[0], t[1]], axis=1)
    h = jnp.maximum(full * dinv + b1_ref[...], 0.0)
    hw = jnp.dot(h, w2_ref[...], preferred_element_type=jnp.float32)
    y2 = hw * dinv
    half = y2.shape[1] // 2
    y2s_ref[0] = y2[:, :half]
    y2s_ref[1] = y2[:, half:]


def _tc3_body(s2s_ref, y2s_ref, dinv_ref, b2_ref, out_ref):
    dinv = dinv_ref[...][:, 0:1]
    t = s2s_ref[...] + y2s_ref[...]
    z = jnp.concatenate([t[0], t[1]], axis=1) * dinv + b2_ref[...]
    m = jnp.max(z, axis=1, keepdims=True)
    e = jnp.exp(z - m)
    out_ref[...] = z - m - jnp.log(jnp.sum(e, axis=1, keepdims=True))


def kernel(x, edge_index, W1, b1, W2, b2):
    n, d_in = x.shape
    e = edge_index.shape[1]
    h_dim = W1.shape[1]
    c_dim = W2.shape[1]
    assert e % NW == 0

    n_acc = (n // 512 + 1) * 512
    n_dummy = n_acc - n

    ept32 = e // NW
    cpt32 = -(-ept32 // (CH * 2 * BG)) * 2 * BG
    ept16 = e // NS
    cpt16 = -(-ept16 // (CH * 2 * BG)) * 2 * BG
    max_pad = max(cpt32 * CH - ept32, cpt16 * CH - ept16)
    pad_src = jnp.zeros((max_pad,), jnp.int32)
    pad_dst = n + jnp.arange(max_pad, dtype=jnp.int32) % n_dummy

    src_flat = edge_index[0]
    dst_flat = edge_index[1]
    x_pad = jnp.concatenate([x, jnp.zeros((n_acc - n, d_in), x.dtype)])

    dw = 8

    degacc = _make_deg_kernel(n_acc, ept32, cpt32, dw)(
        dst_flat, pad_dst, jnp.ones((CH, dw), jnp.float32),
        jnp.zeros((n_acc, dw), jnp.float32),
    )
    xw = pl.pallas_call(
        _tc1a_body,
        out_shape=jax.ShapeDtypeStruct((n_acc, h_dim), jnp.float32),
    )(x_pad, W1)

    dinv, y1 = pl.pallas_call(
        _tc1b_body,
        out_shape=[
            jax.ShapeDtypeStruct((n_acc, 8), jnp.float32),
            jax.ShapeDtypeStruct((NC, n_acc, h_dim // 2), jnp.float32),
        ],
    )(degacc, xw)

    zin_h2 = jnp.zeros((n_acc, h_dim // 2), jnp.float32)
    s1 = _make_colsplit_kernel(n_acc, ept16, cpt16, h_dim)(
        y1, src_flat, dst_flat, pad_src, pad_dst, zin_h2
    )

    y2 = pl.pallas_call(
        _tc2_body,
        out_shape=jax.ShapeDtypeStruct((NC, n_acc, c_dim // 2), jnp.float32),
    )(s1, y1, dinv, W2, b1.reshape(1, h_dim))

    zin_c2 = jnp.zeros((n_acc, c_dim // 2), jnp.float32)
    s2 = _make_colsplit_kernel(n_acc, ept16, cpt16, c_dim)(
        y2, src_flat, dst_flat, pad_src, pad_dst, zin_c2
    )

    out = pl.pallas_call(
        _tc3_body,
        grid=(1,),
        in_specs=[
            pl.BlockSpec((NC, n, c_dim // 2), lambda i: (0, 0, 0)),
            pl.BlockSpec((NC, n, c_dim // 2), lambda i: (0, 0, 0)),
            pl.BlockSpec((n, 8), lambda i: (0, 0)),
            pl.BlockSpec((1, c_dim), lambda i: (0, 0)),
        ],
        out_specs=pl.BlockSpec((n, c_dim), lambda i: (0, 0)),
        out_shape=jax.ShapeDtypeStruct((n, c_dim), jnp.float32),
    )(s2, y2, dinv, b2.reshape(1, c_dim))

    return out


# speedup vs baseline: 1.1325x; 1.1325x over previous
import functools

import jax
import jax.numpy as jnp
from jax import lax
from jax.experimental import pallas as pl
from jax.experimental.pallas import tpu as pltpu
from jax.experimental.pallas import tpu_sc as plsc

NC = 2
NS = 16
NW = NC * NS

CH = 128
BG = 4


def _sc_mesh():
    return plsc.VectorSubcoreMesh(
        core_axis_name="c", subcore_axis_name="s", num_cores=NC, num_subcores=NS
    )


_SC_PARAMS = pltpu.CompilerParams(use_tc_tiling_on_sc=False)


def _zero_fill(ref2d, nrows, width):
    zero = jnp.zeros((16,), jnp.float32)

    def body(j, carry):
        for kk in range(width // 16):
            ref2d[j, pl.ds(kk * 16, 16)] = zero
        return carry

    lax.fori_loop(0, nrows, body, 0)


def _stage_edges(ei_hbm, pad_src_hbm, pad_dst_hbm, src_v, dst_v,
                 base, n_real, n_pad):
    pltpu.sync_copy(ei_hbm.at[0, pl.ds(base, n_real)],
                    src_v.at[pl.ds(0, n_real)])
    pltpu.sync_copy(ei_hbm.at[1, pl.ds(base, n_real)],
                    dst_v.at[pl.ds(0, n_real)])
    pltpu.sync_copy(pad_src_hbm.at[pl.ds(0, n_pad)],
                    src_v.at[pl.ds(n_real, n_pad)])
    pltpu.sync_copy(pad_dst_hbm.at[pl.ds(0, n_pad)],
                    dst_v.at[pl.ds(n_real, n_pad)])


def _make_deg_kernel(n_acc, ept, cpt, dw):
    rows_per_tile = n_acc // NS
    n_pad = cpt * CH - ept

    @functools.partial(
        pl.kernel,
        out_type=jax.ShapeDtypeStruct((NC, n_acc, dw), jnp.float32),
        mesh=_sc_mesh(),
        compiler_params=_SC_PARAMS,
        scratch_types=[
            pltpu.VMEM((cpt * CH,), jnp.int32),
            pltpu.VMEM((CH, dw), jnp.float32),
            pltpu.VMEM_SHARED((n_acc, dw), jnp.float32),
            pltpu.SemaphoreType.DMA,
        ],
    )
    def k(ei_hbm, pad_dst_hbm, ones_hbm, zin_hbm, out_hbm, dst_v, ones_v,
          acc_sh, sem):
        c = lax.axis_index("c")
        s = lax.axis_index("s")
        w = c * NS + s
        pltpu.sync_copy(ei_hbm.at[1, pl.ds(w * ept, ept)],
                        dst_v.at[pl.ds(0, ept)])
        pltpu.sync_copy(pad_dst_hbm.at[pl.ds(0, n_pad)],
                        dst_v.at[pl.ds(ept, n_pad)])
        pltpu.sync_copy(ones_hbm, ones_v)
        pltpu.sync_copy(
            zin_hbm.at[pl.ds(s * rows_per_tile, rows_per_tile)],
            acc_sh.at[pl.ds(s * rows_per_tile, rows_per_tile)],
        )
        plsc.subcore_barrier()

        def fire(j, carry):
            pltpu.async_copy(
                ones_v, acc_sh.at[dst_v.at[pl.ds(j * CH, CH)]], sem, add=True
            )
            return carry

        lax.fori_loop(0, cpt, fire, 0)

        def drain(j, carry):
            pltpu.make_async_copy(
                ones_v, acc_sh.at[dst_v.at[pl.ds(j * CH, CH)]], sem
            ).wait()
            return carry

        lax.fori_loop(0, cpt, drain, 0)
        plsc.subcore_barrier()
        pltpu.sync_copy(
            acc_sh.at[pl.ds(s * rows_per_tile, rows_per_tile)],
            out_hbm.at[c, pl.ds(s * rows_per_tile, rows_per_tile)],
        )

    return k


def _pipelined_edge_loop(y_sh, acc_sh, src_v, dst_v, rows_v, semg, sems,
                         cpt, width):
    nb = cpt // BG
    nbp = nb // 2
    assert cpt % (2 * BG) == 0

    def gslice(h, i):
        return rows_v.at[h, pl.ds(i * CH, CH)]

    def eslice(v, b, i):
        return v.at[pl.ds((b * BG + i) * CH, CH)]

    def issue_gathers(b, h):
        for i in range(BG):
            pltpu.async_copy(y_sh.at[eslice(src_v, b, i)], gslice(h, i),
                             semg[h])

    def wait_gathers(b, h):
        for i in range(BG):
            pltpu.make_async_copy(y_sh.at[eslice(src_v, b, i)], gslice(h, i),
                                  semg[h]).wait()

    def issue_scatters(b, h):
        for i in range(BG):
            pltpu.async_copy(gslice(h, i), acc_sh.at[eslice(dst_v, b, i)],
                             sems[h], add=True)

    def wait_scatters(b, h):
        for i in range(BG):
            pltpu.make_async_copy(gslice(h, i), acc_sh.at[eslice(dst_v, b, i)],
                                  sems[h]).wait()

    issue_gathers(0, 0)

    def body(bp, carry):
        b0 = 2 * bp
        wait_gathers(b0, 0)
        issue_scatters(b0, 0)

        @pl.when(bp > 0)
        def _():
            wait_scatters(b0 - 1, 1)

        issue_gathers(b0 + 1, 1)
        wait_gathers(b0 + 1, 1)
        issue_scatters(b0 + 1, 1)

        @pl.when(bp < nbp - 1)
        def _():
            wait_scatters(b0, 0)
            issue_gathers(b0 + 2, 0)

        return carry

    lax.fori_loop(0, nbp, body, 0)
    wait_scatters(nb - 2, 0)
    wait_scatters(nb - 1, 1)


def _make_colsplit_kernel(n_acc, ept, cpt, width):
    rows_per_tile = n_acc // NS
    colw = width // 2
    n_pad = cpt * CH - ept

    @functools.partial(
        pl.kernel,
        out_type=jax.ShapeDtypeStruct((n_acc, width), jnp.float32),
        mesh=_sc_mesh(),
        compiler_params=_SC_PARAMS,
        scratch_types=[
            pltpu.VMEM((cpt * CH,), jnp.int32),
            pltpu.VMEM((cpt * CH,), jnp.int32),
            pltpu.VMEM((2, BG * CH, colw), jnp.float32),
            pltpu.VMEM_SHARED((n_acc, colw), jnp.float32),
            pltpu.VMEM_SHARED((n_acc, colw), jnp.float32),
            pltpu.SemaphoreType.DMA,
            pltpu.SemaphoreType.DMA,
            pltpu.SemaphoreType.DMA,
            pltpu.SemaphoreType.DMA,
        ],
    )
    def k(y_hbm, ei_hbm, pad_src_hbm, pad_dst_hbm, zin_hbm, out_hbm,
          src_v, dst_v, rows_v, acc_sh, y_sh, semg0, semg1, sems0, sems1):
        c = lax.axis_index("c")
        s = lax.axis_index("s")
        _stage_edges(ei_hbm, pad_src_hbm, pad_dst_hbm,
                     src_v, dst_v, s * ept, ept, n_pad)
        pltpu.sync_copy(
            y_hbm.at[pl.ds(s * rows_per_tile, rows_per_tile),
                     pl.ds(c * colw, colw)],
            y_sh.at[pl.ds(s * rows_per_tile, rows_per_tile)],
        )
        pltpu.sync_copy(
            zin_hbm.at[pl.ds(s * rows_per_tile, rows_per_tile)],
            acc_sh.at[pl.ds(s * rows_per_tile, rows_per_tile)],
        )
        plsc.subcore_barrier()
        _pipelined_edge_loop(y_sh, acc_sh, src_v, dst_v, rows_v,
                             (semg0, semg1), (sems0, sems1), cpt, colw)
        plsc.subcore_barrier()
        pltpu.sync_copy(
            acc_sh.at[pl.ds(s * rows_per_tile, rows_per_tile)],
            out_hbm.at[pl.ds(s * rows_per_tile, rows_per_tile),
                       pl.ds(c * colw, colw)],
        )

    return k


def _tc1a_body(x_ref, w1_ref, xw_ref):
    xw_ref[...] = jnp.dot(
        x_ref[...], w1_ref[...], preferred_element_type=jnp.float32
    )


def _tc1b_body(degacc_ref, xw_ref, dinv_ref, y1_ref):
    d = degacc_ref[...]
    deg = d[0, :, 0:1] + d[1, :, 0:1] + 1.0
    dinv = lax.rsqrt(deg)
    y1_ref[...] = xw_ref[...] * dinv
    dinv_ref[...] = jnp.broadcast_to(dinv, dinv_ref.shape)


def _tc2_body(s1_ref, y1_ref, dinv_ref, w2_ref, b1_ref, y2_ref):
    dinv = dinv_ref[...][:, 0:1]
    h = jnp.maximum((s1_ref[...] + y1_ref[...]) * dinv + b1_ref[...], 0.0)
    hw = jnp.dot(h, w2_ref[...], preferred_element_type=jnp.float32)
    y2_ref[...] = hw * dinv


def _tc3_body(s2_ref, y2_ref, dinv_ref, b2_ref, out_ref):
    dinv = dinv_ref[...][:, 0:1]
    z = (s2_ref[...] + y2_ref[...]) * dinv + b2_ref[...]
    m = jnp.max(z, axis=1, keepdims=True)
    e = jnp.exp(z - m)
    out_ref[...] = z - m - jnp.log(jnp.sum(e, axis=1, keepdims=True))


def kernel(x, edge_index, W1, b1, W2, b2):
    n, d_in = x.shape
    e = edge_index.shape[1]
    h_dim = W1.shape[1]
    c_dim = W2.shape[1]
    assert e % NW == 0

    n_acc = (n // 512 + 1) * 512
    n_dummy = n_acc - n

    ept32 = e // NW
    cpt32 = -(-ept32 // (CH * 2 * BG)) * 2 * BG
    ept16 = e // NS
    cpt16 = -(-ept16 // (CH * 2 * BG)) * 2 * BG
    max_pad = max(cpt32 * CH - ept32, cpt16 * CH - ept16)
    pad_src = jnp.zeros((max_pad,), jnp.int32)
    pad_dst = n + jnp.arange(max_pad, dtype=jnp.int32) % n_dummy

    x_pad = jnp.concatenate([x, jnp.zeros((n_acc - n, d_in), x.dtype)])

    dw = 8

    degacc = _make_deg_kernel(n_acc, ept32, cpt32, dw)(
        edge_index, pad_dst, jnp.ones((CH, dw), jnp.float32),
        jnp.zeros((n_acc, dw), jnp.float32),
    )
    xw = pl.pallas_call(
        _tc1a_body,
        out_shape=jax.ShapeDtypeStruct((n_acc, h_dim), jnp.float32),
    )(x_pad, W1)

    dinv, y1 = pl.pallas_call(
        _tc1b_body,
        out_shape=[
            jax.ShapeDtypeStruct((n_acc, 8), jnp.float32),
            jax.ShapeDtypeStruct((n_acc, h_dim), jnp.float32),
        ],
    )(degacc, xw)

    zin_h2 = jnp.zeros((n_acc, h_dim // 2), jnp.float32)
    s1 = _make_colsplit_kernel(n_acc, ept16, cpt16, h_dim)(
        y1, edge_index, pad_src, pad_dst, zin_h2
    )

    y2 = pl.pallas_call(
        _tc2_body,
        out_shape=jax.ShapeDtypeStruct((n_acc, c_dim), jnp.float32),
    )(s1, y1, dinv, W2, b1.reshape(1, h_dim))

    zin_c2 = jnp.zeros((n_acc, c_dim // 2), jnp.float32)
    s2 = _make_colsplit_kernel(n_acc, ept16, cpt16, c_dim)(
        y2, edge_index, pad_src, pad_dst, zin_c2
    )

    out = pl.pallas_call(
        _tc3_body,
        grid=(1,),
        in_specs=[
            pl.BlockSpec((n, c_dim), lambda i: (0, 0)),
            pl.BlockSpec((n, c_dim), lambda i: (0, 0)),
            pl.BlockSpec((n, 8), lambda i: (0, 0)),
            pl.BlockSpec((1, c_dim), lambda i: (0, 0)),
        ],
        out_specs=pl.BlockSpec((n, c_dim), lambda i: (0, 0)),
        out_shape=jax.ShapeDtypeStruct((n, c_dim), jnp.float32),
    )(s2, y2, dinv, b2.reshape(1, c_dim))

    return out


# speedup vs baseline: 1.1617x; 1.0258x over previous
import functools

import jax
import jax.numpy as jnp
from jax import lax
from jax.experimental import pallas as pl
from jax.experimental.pallas import tpu as pltpu
from jax.experimental.pallas import tpu_sc as plsc

NC = 2
NS = 16
NW = NC * NS

CH = 128
BG = 4


def _sc_mesh():
    return plsc.VectorSubcoreMesh(
        core_axis_name="c", subcore_axis_name="s", num_cores=NC, num_subcores=NS
    )


_SC_PARAMS = pltpu.CompilerParams(use_tc_tiling_on_sc=False)


def _zero_fill(ref2d, nrows, width):
    zero = jnp.zeros((16,), jnp.float32)

    def body(j, carry):
        for kk in range(width // 16):
            ref2d[j, pl.ds(kk * 16, 16)] = zero
        return carry

    lax.fori_loop(0, nrows, body, 0)


def _stage_edges(ei_hbm, pad_src_hbm, pad_dst_hbm, src_v, dst_v,
                 base, n_real, n_pad):
    pltpu.sync_copy(ei_hbm.at[0, pl.ds(base, n_real)],
                    src_v.at[pl.ds(0, n_real)])
    pltpu.sync_copy(ei_hbm.at[1, pl.ds(base, n_real)],
                    dst_v.at[pl.ds(0, n_real)])
    pltpu.sync_copy(pad_src_hbm.at[pl.ds(0, n_pad)],
                    src_v.at[pl.ds(n_real, n_pad)])
    pltpu.sync_copy(pad_dst_hbm.at[pl.ds(0, n_pad)],
                    dst_v.at[pl.ds(n_real, n_pad)])


def _make_deg_kernel(n_acc, ept, cpt, dw):
    rows_per_tile = n_acc // NS
    n_pad = cpt * CH - ept

    @functools.partial(
        pl.kernel,
        out_type=jax.ShapeDtypeStruct((NC, n_acc, dw), jnp.float32),
        mesh=_sc_mesh(),
        compiler_params=_SC_PARAMS,
        scratch_types=[
            pltpu.VMEM((cpt * CH,), jnp.int32),
            pltpu.VMEM((CH, dw), jnp.float32),
            pltpu.VMEM_SHARED((n_acc, dw), jnp.float32),
            pltpu.SemaphoreType.DMA,
        ],
    )
    def k(ei_hbm, pad_dst_hbm, ones_hbm, zin_hbm, out_hbm, dst_v, ones_v,
          acc_sh, sem):
        c = lax.axis_index("c")
        s = lax.axis_index("s")
        w = c * NS + s
        pltpu.sync_copy(ei_hbm.at[1, pl.ds(w * ept, ept)],
                        dst_v.at[pl.ds(0, ept)])
        pltpu.sync_copy(pad_dst_hbm.at[pl.ds(0, n_pad)],
                        dst_v.at[pl.ds(ept, n_pad)])
        pltpu.sync_copy(ones_hbm, ones_v)
        pltpu.sync_copy(
            zin_hbm.at[pl.ds(s * rows_per_tile, rows_per_tile)],
            acc_sh.at[pl.ds(s * rows_per_tile, rows_per_tile)],
        )
        plsc.subcore_barrier()

        def fire(j, carry):
            pltpu.async_copy(
                ones_v, acc_sh.at[dst_v.at[pl.ds(j * CH, CH)]], sem, add=True
            )
            return carry

        lax.fori_loop(0, cpt, fire, 0)

        def drain(j, carry):
            pltpu.make_async_copy(
                ones_v, acc_sh.at[dst_v.at[pl.ds(j * CH, CH)]], sem
            ).wait()
            return carry

        lax.fori_loop(0, cpt, drain, 0)
        plsc.subcore_barrier()
        pltpu.sync_copy(
            acc_sh.at[pl.ds(s * rows_per_tile, rows_per_tile)],
            out_hbm.at[c, pl.ds(s * rows_per_tile, rows_per_tile)],
        )

    return k


def _pipelined_edge_loop(y_sh, acc_sh, src_v, dst_v, rows_v, semg, sems,
                         cpt, width):
    nb = cpt // BG
    nbp = nb // 2
    assert cpt % (2 * BG) == 0

    def gslice(h, i):
        return rows_v.at[h, pl.ds(i * CH, CH)]

    def eslice(v, b, i):
        return v.at[pl.ds((b * BG + i) * CH, CH)]

    def issue_gathers(b, h):
        for i in range(BG):
            pltpu.async_copy(y_sh.at[eslice(src_v, b, i)], gslice(h, i),
                             semg[h])

    def wait_gathers(b, h):
        for i in range(BG):
            pltpu.make_async_copy(y_sh.at[eslice(src_v, b, i)], gslice(h, i),
                                  semg[h]).wait()

    def issue_scatters(b, h):
        for i in range(BG):
            pltpu.async_copy(gslice(h, i), acc_sh.at[eslice(dst_v, b, i)],
                             sems[h], add=True)

    def wait_scatters(b, h):
        for i in range(BG):
            pltpu.make_async_copy(gslice(h, i), acc_sh.at[eslice(dst_v, b, i)],
                                  sems[h]).wait()

    issue_gathers(0, 0)

    def body(bp, carry):
        b0 = 2 * bp
        wait_gathers(b0, 0)
        issue_scatters(b0, 0)

        @pl.when(bp > 0)
        def _():
            wait_scatters(b0 - 1, 1)

        issue_gathers(b0 + 1, 1)
        wait_gathers(b0 + 1, 1)
        issue_scatters(b0 + 1, 1)

        @pl.when(bp < nbp - 1)
        def _():
            wait_scatters(b0, 0)
            issue_gathers(b0 + 2, 0)

        return carry

    lax.fori_loop(0, nbp, body, 0)
    wait_scatters(nb - 2, 0)
    wait_scatters(nb - 1, 1)


def _make_colsplit_kernel(n_acc, ept, cpt, width):
    rows_per_tile = n_acc // NS
    colw = width // 2
    n_pad = cpt * CH - ept

    @functools.partial(
        pl.kernel,
        out_type=jax.ShapeDtypeStruct((n_acc, width), jnp.float32),
        mesh=_sc_mesh(),
        compiler_params=_SC_PARAMS,
        scratch_types=[
            pltpu.VMEM((cpt * CH,), jnp.int32),
            pltpu.VMEM((cpt * CH,), jnp.int32),
            pltpu.VMEM((2, BG * CH, colw), jnp.float32),
            pltpu.VMEM_SHARED((n_acc, colw), jnp.float32),
            pltpu.VMEM_SHARED((n_acc, colw), jnp.float32),
            pltpu.SemaphoreType.DMA,
            pltpu.SemaphoreType.DMA,
            pltpu.SemaphoreType.DMA,
            pltpu.SemaphoreType.DMA,
        ],
    )
    def k(y_hbm, ei_hbm, pad_src_hbm, pad_dst_hbm, zin_hbm, out_hbm,
          src_v, dst_v, rows_v, acc_sh, y_sh, semg0, semg1, sems0, sems1):
        c = lax.axis_index("c")
        s = lax.axis_index("s")
        _stage_edges(ei_hbm, pad_src_hbm, pad_dst_hbm,
                     src_v, dst_v, s * ept, ept, n_pad)
        pltpu.sync_copy(
            y_hbm.at[pl.ds(s * rows_per_tile, rows_per_tile),
                     pl.ds(c * colw, colw)],
            y_sh.at[pl.ds(s * rows_per_tile, rows_per_tile)],
        )
        pltpu.sync_copy(
            zin_hbm.at[pl.ds(s * rows_per_tile, rows_per_tile)],
            acc_sh.at[pl.ds(s * rows_per_tile, rows_per_tile)],
        )
        plsc.subcore_barrier()
        _pipelined_edge_loop(y_sh, acc_sh, src_v, dst_v, rows_v,
                             (semg0, semg1), (sems0, sems1), cpt, colw)
        plsc.subcore_barrier()
        pltpu.sync_copy(
            acc_sh.at[pl.ds(s * rows_per_tile, rows_per_tile)],
            out_hbm.at[pl.ds(s * rows_per_tile, rows_per_tile),
                       pl.ds(c * colw, colw)],
        )

    return k


def _make_scatter_kernel(n_acc, ept, cpt, width):
    rows_per_tile = n_acc // NS
    n_pad = cpt * CH - ept

    @functools.partial(
        pl.kernel,
        out_type=jax.ShapeDtypeStruct((NC, n_acc, width), jnp.float32),
        mesh=_sc_mesh(),
        compiler_params=_SC_PARAMS,
        scratch_types=[
            pltpu.VMEM((cpt * CH,), jnp.int32),
            pltpu.VMEM((cpt * CH,), jnp.int32),
            pltpu.VMEM((2, BG * CH, width), jnp.float32),
            pltpu.VMEM_SHARED((n_acc, width), jnp.float32),
            pltpu.VMEM_SHARED((n_acc, width), jnp.float32),
            pltpu.SemaphoreType.DMA,
            pltpu.SemaphoreType.DMA,
            pltpu.SemaphoreType.DMA,
            pltpu.SemaphoreType.DMA,
        ],
    )
    def k(y_hbm, ei_hbm, pad_src_hbm, pad_dst_hbm, zin_hbm, out_hbm,
          src_v, dst_v, rows_v, acc_sh, y_sh, semg0, semg1, sems0, sems1):
        c = lax.axis_index("c")
        s = lax.axis_index("s")
        w = c * NS + s
        _stage_edges(ei_hbm, pad_src_hbm, pad_dst_hbm,
                     src_v, dst_v, w * ept, ept, n_pad)
        pltpu.sync_copy(
            y_hbm.at[pl.ds(s * rows_per_tile, rows_per_tile)],
            y_sh.at[pl.ds(s * rows_per_tile, rows_per_tile)],
        )
        pltpu.sync_copy(
            zin_hbm.at[pl.ds(s * rows_per_tile, rows_per_tile)],
            acc_sh.at[pl.ds(s * rows_per_tile, rows_per_tile)],
        )
        plsc.subcore_barrier()
        _pipelined_edge_loop(y_sh, acc_sh, src_v, dst_v, rows_v,
                             (semg0, semg1), (sems0, sems1), cpt, width)
        plsc.subcore_barrier()
        pltpu.sync_copy(
            acc_sh.at[pl.ds(s * rows_per_tile, rows_per_tile)],
            out_hbm.at[c, pl.ds(s * rows_per_tile, rows_per_tile)],
        )

    return k


def _tc1a_body(x_ref, w1_ref, xw_ref):
    xw_ref[...] = jnp.dot(
        x_ref[...], w1_ref[...], preferred_element_type=jnp.float32
    )


def _tc1b_body(degacc_ref, xw_ref, dinv_ref, y1_ref):
    d = degacc_ref[...]
    deg = d[0, :, 0:1] + d[1, :, 0:1] + 1.0
    dinv = lax.rsqrt(deg)
    y1_ref[...] = xw_ref[...] * dinv
    dinv_ref[...] = jnp.broadcast_to(dinv, dinv_ref.shape)


def _tc2_body(s1_ref, y1_ref, dinv_ref, w2_ref, b1_ref, y2_ref):
    dinv = dinv_ref[...][:, 0:1]
    h = jnp.maximum((s1_ref[...] + y1_ref[...]) * dinv + b1_ref[...], 0.0)
    hw = jnp.dot(h, w2_ref[...], preferred_element_type=jnp.float32)
    y2_ref[...] = hw * dinv


def _tc3_body(s2_ref, y2_ref, dinv_ref, b2_ref, out_ref):
    s2 = s2_ref[...]
    dinv = dinv_ref[...][:, 0:1]
    z = (s2[0] + s2[1] + y2_ref[...]) * dinv + b2_ref[...]
    m = jnp.max(z, axis=1, keepdims=True)
    e = jnp.exp(z - m)
    out_ref[...] = z - m - jnp.log(jnp.sum(e, axis=1, keepdims=True))


def kernel(x, edge_index, W1, b1, W2, b2):
    n, d_in = x.shape
    e = edge_index.shape[1]
    h_dim = W1.shape[1]
    c_dim = W2.shape[1]
    assert e % NW == 0

    n_acc = (n // 512 + 1) * 512
    n_dummy = n_acc - n

    ept32 = e // NW
    cpt32 = -(-ept32 // (CH * 2 * BG)) * 2 * BG
    ept16 = e // NS
    cpt16 = -(-ept16 // (CH * 2 * BG)) * 2 * BG
    max_pad = max(cpt32 * CH - ept32, cpt16 * CH - ept16)
    pad_src = jnp.zeros((max_pad,), jnp.int32)
    pad_dst = n + jnp.arange(max_pad, dtype=jnp.int32) % n_dummy

    x_pad = jnp.concatenate([x, jnp.zeros((n_acc - n, d_in), x.dtype)])

    dw = 8

    degacc = _make_deg_kernel(n_acc, ept32, cpt32, dw)(
        edge_index, pad_dst, jnp.ones((CH, dw), jnp.float32),
        jnp.zeros((n_acc, dw), jnp.float32),
    )
    xw = pl.pallas_call(
        _tc1a_body,
        out_shape=jax.ShapeDtypeStruct((n_acc, h_dim), jnp.float32),
    )(x_pad, W1)

    dinv, y1 = pl.pallas_call(
        _tc1b_body,
        out_shape=[
            jax.ShapeDtypeStruct((n_acc, 8), jnp.float32),
            jax.ShapeDtypeStruct((n_acc, h_dim), jnp.float32),
        ],
    )(degacc, xw)

    zin_h2 = jnp.zeros((n_acc, h_dim // 2), jnp.float32)
    s1 = _make_colsplit_kernel(n_acc, ept16, cpt16, h_dim)(
        y1, edge_index, pad_src, pad_dst, zin_h2
    )

    y2 = pl.pallas_call(
        _tc2_body,
        out_shape=jax.ShapeDtypeStruct((n_acc, c_dim), jnp.float32),
    )(s1, y1, dinv, W2, b1.reshape(1, h_dim))

    zin_c = jnp.zeros((n_acc, c_dim), jnp.float32)
    s2 = _make_scatter_kernel(n_acc, ept32, cpt32, c_dim)(
        y2, edge_index, pad_src, pad_dst, zin_c
    )

    out = pl.pallas_call(
        _tc3_body,
        grid=(1,),
        in_specs=[
            pl.BlockSpec((NC, n, c_dim), lambda i: (0, 0, 0)),
            pl.BlockSpec((n, c_dim), lambda i: (0, 0)),
            pl.BlockSpec((n, 8), lambda i: (0, 0)),
            pl.BlockSpec((1, c_dim), lambda i: (0, 0)),
        ],
        out_specs=pl.BlockSpec((n, c_dim), lambda i: (0, 0)),
        out_shape=jax.ShapeDtypeStruct((n, c_dim), jnp.float32),
    )(s2, y2, dinv, b2.reshape(1, c_dim))

    return out


# speedup vs baseline: 1.1689x; 1.0062x over previous
import functools

import jax
import jax.numpy as jnp
from jax import lax
from jax.experimental import pallas as pl
from jax.experimental.pallas import tpu as pltpu
from jax.experimental.pallas import tpu_sc as plsc

NC = 2
NS = 16
NW = NC * NS

CH = 128
BG = 5


def _sc_mesh():
    return plsc.VectorSubcoreMesh(
        core_axis_name="c", subcore_axis_name="s", num_cores=NC, num_subcores=NS
    )


_SC_PARAMS = pltpu.CompilerParams(use_tc_tiling_on_sc=False)


def _zero_fill(ref2d, nrows, width):
    zero = jnp.zeros((16,), jnp.float32)

    def body(j, carry):
        for kk in range(width // 16):
            ref2d[j, pl.ds(kk * 16, 16)] = zero
        return carry

    lax.fori_loop(0, nrows, body, 0)


def _stage_edges(ei_hbm, pad_src_hbm, pad_dst_hbm, src_v, dst_v,
                 base, n_real, n_pad):
    pltpu.sync_copy(ei_hbm.at[0, pl.ds(base, n_real)],
                    src_v.at[pl.ds(0, n_real)])
    pltpu.sync_copy(ei_hbm.at[1, pl.ds(base, n_real)],
                    dst_v.at[pl.ds(0, n_real)])
    pltpu.sync_copy(pad_src_hbm.at[pl.ds(0, n_pad)],
                    src_v.at[pl.ds(n_real, n_pad)])
    pltpu.sync_copy(pad_dst_hbm.at[pl.ds(0, n_pad)],
                    dst_v.at[pl.ds(n_real, n_pad)])


def _make_deg_kernel(n_acc, ept, cpt, dw):
    rows_per_tile = n_acc // NS
    n_pad = cpt * CH - ept

    @functools.partial(
        pl.kernel,
        out_type=jax.ShapeDtypeStruct((NC, n_acc, dw), jnp.float32),
        mesh=_sc_mesh(),
        compiler_params=_SC_PARAMS,
        scratch_types=[
            pltpu.VMEM((cpt * CH,), jnp.int32),
            pltpu.VMEM((CH, dw), jnp.float32),
            pltpu.VMEM_SHARED((n_acc, dw), jnp.float32),
            pltpu.SemaphoreType.DMA,
        ],
    )
    def k(ei_hbm, pad_dst_hbm, ones_hbm, zin_hbm, out_hbm, dst_v, ones_v,
          acc_sh, sem):
        c = lax.axis_index("c")
        s = lax.axis_index("s")
        w = c * NS + s
        pltpu.sync_copy(ei_hbm.at[1, pl.ds(w * ept, ept)],
                        dst_v.at[pl.ds(0, ept)])
        pltpu.sync_copy(pad_dst_hbm.at[pl.ds(0, n_pad)],
                        dst_v.at[pl.ds(ept, n_pad)])
        pltpu.sync_copy(ones_hbm, ones_v)
        pltpu.sync_copy(
            zin_hbm.at[pl.ds(s * rows_per_tile, rows_per_tile)],
            acc_sh.at[pl.ds(s * rows_per_tile, rows_per_tile)],
        )
        plsc.subcore_barrier()

        def fire(j, carry):
            pltpu.async_copy(
                ones_v, acc_sh.at[dst_v.at[pl.ds(j * CH, CH)]], sem, add=True
            )
            return carry

        lax.fori_loop(0, cpt, fire, 0)

        def drain(j, carry):
            pltpu.make_async_copy(
                ones_v, acc_sh.at[dst_v.at[pl.ds(j * CH, CH)]], sem
            ).wait()
            return carry

        lax.fori_loop(0, cpt, drain, 0)
        plsc.subcore_barrier()
        pltpu.sync_copy(
            acc_sh.at[pl.ds(s * rows_per_tile, rows_per_tile)],
            out_hbm.at[c, pl.ds(s * rows_per_tile, rows_per_tile)],
        )

    return k


def _pipelined_edge_loop(y_sh, acc_sh, src_v, dst_v, rows_v, semg, sems,
                         cpt, width):
    nb = cpt // BG
    nbp = nb // 2
    assert cpt % (2 * BG) == 0

    def gslice(h, i):
        return rows_v.at[h, pl.ds(i * CH, CH)]

    def eslice(v, b, i):
        return v.at[pl.ds((b * BG + i) * CH, CH)]

    def issue_gathers(b, h):
        for i in range(BG):
            pltpu.async_copy(y_sh.at[eslice(src_v, b, i)], gslice(h, i),
                             semg[h])

    def wait_gathers(b, h):
        for i in range(BG):
            pltpu.make_async_copy(y_sh.at[eslice(src_v, b, i)], gslice(h, i),
                                  semg[h]).wait()

    def issue_scatters(b, h):
        for i in range(BG):
            pltpu.async_copy(gslice(h, i), acc_sh.at[eslice(dst_v, b, i)],
                             sems[h], add=True)

    def wait_scatters(b, h):
        for i in range(BG):
            pltpu.make_async_copy(gslice(h, i), acc_sh.at[eslice(dst_v, b, i)],
                                  sems[h]).wait()

    issue_gathers(0, 0)

    def body(bp, carry):
        b0 = 2 * bp
        wait_gathers(b0, 0)
        issue_scatters(b0, 0)

        @pl.when(bp > 0)
        def _():
            wait_scatters(b0 - 1, 1)

        issue_gathers(b0 + 1, 1)
        wait_gathers(b0 + 1, 1)
        issue_scatters(b0 + 1, 1)

        @pl.when(bp < nbp - 1)
        def _():
            wait_scatters(b0, 0)
            issue_gathers(b0 + 2, 0)

        return carry

    lax.fori_loop(0, nbp, body, 0)
    wait_scatters(nb - 2, 0)
    wait_scatters(nb - 1, 1)


def _make_colsplit_kernel(n_acc, ept, cpt, width):
    rows_per_tile = n_acc // NS
    colw = width // 2
    n_pad = cpt * CH - ept

    @functools.partial(
        pl.kernel,
        out_type=jax.ShapeDtypeStruct((n_acc, width), jnp.float32),
        mesh=_sc_mesh(),
        compiler_params=_SC_PARAMS,
        scratch_types=[
            pltpu.VMEM((cpt * CH,), jnp.int32),
            pltpu.VMEM((cpt * CH,), jnp.int32),
            pltpu.VMEM((2, BG * CH, colw), jnp.float32),
            pltpu.VMEM_SHARED((n_acc, colw), jnp.float32),
            pltpu.VMEM_SHARED((n_acc, colw), jnp.float32),
            pltpu.SemaphoreType.DMA,
            pltpu.SemaphoreType.DMA,
            pltpu.SemaphoreType.DMA,
            pltpu.SemaphoreType.DMA,
        ],
    )
    def k(y_hbm, ei_hbm, pad_src_hbm, pad_dst_hbm, zin_hbm, out_hbm,
          src_v, dst_v, rows_v, acc_sh, y_sh, semg0, semg1, sems0, sems1):
        c = lax.axis_index("c")
        s = lax.axis_index("s")
        _stage_edges(ei_hbm, pad_src_hbm, pad_dst_hbm,
                     src_v, dst_v, s * ept, ept, n_pad)
        pltpu.sync_copy(
            y_hbm.at[pl.ds(s * rows_per_tile, rows_per_tile),
                     pl.ds(c * colw, colw)],
            y_sh.at[pl.ds(s * rows_per_tile, rows_per_tile)],
        )
        pltpu.sync_copy(
            zin_hbm.at[pl.ds(s * rows_per_tile, rows_per_tile)],
            acc_sh.at[pl.ds(s * rows_per_tile, rows_per_tile)],
        )
        plsc.subcore_barrier()
        _pipelined_edge_loop(y_sh, acc_sh, src_v, dst_v, rows_v,
                             (semg0, semg1), (sems0, sems1), cpt, colw)
        plsc.subcore_barrier()
        pltpu.sync_copy(
            acc_sh.at[pl.ds(s * rows_per_tile, rows_per_tile)],
            out_hbm.at[pl.ds(s * rows_per_tile, rows_per_tile),
                       pl.ds(c * colw, colw)],
        )

    return k


def _make_scatter_kernel(n_acc, ept, cpt, width):
    rows_per_tile = n_acc // NS
    n_pad = cpt * CH - ept

    @functools.partial(
        pl.kernel,
        out_type=jax.ShapeDtypeStruct((NC, n_acc, width), jnp.float32),
        mesh=_sc_mesh(),
        compiler_params=_SC_PARAMS,
        scratch_types=[
            pltpu.VMEM((cpt * CH,), jnp.int32),
            pltpu.VMEM((cpt * CH,), jnp.int32),
            pltpu.VMEM((2, BG * CH, width), jnp.float32),
            pltpu.VMEM_SHARED((n_acc, width), jnp.float32),
            pltpu.VMEM_SHARED((n_acc, width), jnp.float32),
            pltpu.SemaphoreType.DMA,
            pltpu.SemaphoreType.DMA,
            pltpu.SemaphoreType.DMA,
            pltpu.SemaphoreType.DMA,
        ],
    )
    def k(y_hbm, ei_hbm, pad_src_hbm, pad_dst_hbm, zin_hbm, out_hbm,
          src_v, dst_v, rows_v, acc_sh, y_sh, semg0, semg1, sems0, sems1):
        c = lax.axis_index("c")
        s = lax.axis_index("s")
        w = c * NS + s
        _stage_edges(ei_hbm, pad_src_hbm, pad_dst_hbm,
                     src_v, dst_v, w * ept, ept, n_pad)
        pltpu.sync_copy(
            y_hbm.at[pl.ds(s * rows_per_tile, rows_per_tile)],
            y_sh.at[pl.ds(s * rows_per_tile, rows_per_tile)],
        )
        pltpu.sync_copy(
            zin_hbm.at[pl.ds(s * rows_per_tile, rows_per_tile)],
            acc_sh.at[pl.ds(s * rows_per_tile, rows_per_tile)],
        )
        plsc.subcore_barrier()
        _pipelined_edge_loop(y_sh, acc_sh, src_v, dst_v, rows_v,
                             (semg0, semg1), (sems0, sems1), cpt, width)
        plsc.subcore_barrier()
        pltpu.sync_copy(
            acc_sh.at[pl.ds(s * rows_per_tile, rows_per_tile)],
            out_hbm.at[c, pl.ds(s * rows_per_tile, rows_per_tile)],
        )

    return k


def _tc1a_body(x_ref, w1_ref, xw_ref):
    xw_ref[...] = jnp.dot(
        x_ref[...], w1_ref[...], preferred_element_type=jnp.float32
    )


def _tc1b_body(degacc_ref, xw_ref, dinv_ref, y1_ref):
    d = degacc_ref[...]
    deg = d[0, :, 0:1] + d[1, :, 0:1] + 1.0
    dinv = lax.rsqrt(deg)
    y1_ref[...] = xw_ref[...] * dinv
    dinv_ref[...] = jnp.broadcast_to(dinv, dinv_ref.shape)


def _tc2_body(s1_ref, y1_ref, dinv_ref, w2_ref, b1_ref, y2_ref):
    dinv = dinv_ref[...][:, 0:1]
    h = jnp.maximum((s1_ref[...] + y1_ref[...]) * dinv + b1_ref[...], 0.0)
    hw = jnp.dot(h, w2_ref[...], preferred_element_type=jnp.float32)
    y2_ref[...] = hw * dinv


def _tc3_body(s2_ref, y2_ref, dinv_ref, b2_ref, out_ref):
    s2 = s2_ref[...]
    dinv = dinv_ref[...][:, 0:1]
    z = (s2[0] + s2[1] + y2_ref[...]) * dinv + b2_ref[...]
    m = jnp.max(z, axis=1, keepdims=True)
    e = jnp.exp(z - m)
    out_ref[...] = z - m - jnp.log(jnp.sum(e, axis=1, keepdims=True))


def kernel(x, edge_index, W1, b1, W2, b2):
    n, d_in = x.shape
    e = edge_index.shape[1]
    h_dim = W1.shape[1]
    c_dim = W2.shape[1]
    assert e % NW == 0

    n_acc = (n // 512 + 1) * 512
    n_dummy = n_acc - n

    ept32 = e // NW
    cpt32 = -(-ept32 // (CH * 2 * BG)) * 2 * BG
    ept16 = e // NS
    cpt16 = -(-ept16 // (CH * 2 * BG)) * 2 * BG
    max_pad = max(cpt32 * CH - ept32, cpt16 * CH - ept16)
    pad_src = jnp.zeros((max_pad,), jnp.int32)
    pad_dst = n + jnp.arange(max_pad, dtype=jnp.int32) % n_dummy

    x_pad = jnp.concatenate([x, jnp.zeros((n_acc - n, d_in), x.dtype)])

    dw = 8

    degacc = _make_deg_kernel(n_acc, ept32, cpt32, dw)(
        edge_index, pad_dst, jnp.ones((CH, dw), jnp.float32),
        jnp.zeros((n_acc, dw), jnp.float32),
    )
    xw = pl.pallas_call(
        _tc1a_body,
        out_shape=jax.ShapeDtypeStruct((n_acc, h_dim), jnp.float32),
    )(x_pad, W1)

    dinv, y1 = pl.pallas_call(
        _tc1b_body,
        out_shape=[
            jax.ShapeDtypeStruct((n_acc, 8), jnp.float32),
            jax.ShapeDtypeStruct((n_acc, h_dim), jnp.float32),
        ],
    )(degacc, xw)

    zin_h2 = jnp.zeros((n_acc, h_dim // 2), jnp.float32)
    s1 = _make_colsplit_kernel(n_acc, ept16, cpt16, h_dim)(
        y1, edge_index, pad_src, pad_dst, zin_h2
    )

    y2 = pl.pallas_call(
        _tc2_body,
        out_shape=jax.ShapeDtypeStruct((n_acc, c_dim), jnp.float32),
    )(s1, y1, dinv, W2, b1.reshape(1, h_dim))

    zin_c = jnp.zeros((n_acc, c_dim), jnp.float32)
    s2 = _make_scatter_kernel(n_acc, ept32, cpt32, c_dim)(
        y2, edge_index, pad_src, pad_dst, zin_c
    )

    out = pl.pallas_call(
        _tc3_body,
        grid=(1,),
        in_specs=[
            pl.BlockSpec((NC, n, c_dim), lambda i: (0, 0, 0)),
            pl.BlockSpec((n, c_dim), lambda i: (0, 0)),
            pl.BlockSpec((n, 8), lambda i: (0, 0)),
            pl.BlockSpec((1, c_dim), lambda i: (0, 0)),
        ],
        out_specs=pl.BlockSpec((n, c_dim), lambda i: (0, 0)),
        out_shape=jax.ShapeDtypeStruct((n, c_dim), jnp.float32),
    )(s2, y2, dinv, b2.reshape(1, c_dim))

    return out


# speedup vs baseline: 1.2526x; 1.0717x over previous
import functools

import jax
import jax.numpy as jnp
from jax import lax
from jax.experimental import pallas as pl
from jax.experimental.pallas import tpu as pltpu
from jax.experimental.pallas import tpu_sc as plsc

NC = 2
NS = 16
NW = NC * NS

CH = 128
BG = 4


def _sc_mesh():
    return plsc.VectorSubcoreMesh(
        core_axis_name="c", subcore_axis_name="s", num_cores=NC, num_subcores=NS
    )


_SC_PARAMS = pltpu.CompilerParams(
    use_tc_tiling_on_sc=False, needs_layout_passes=False
)


def _zero_fill(ref2d, nrows, width):
    zero = jnp.zeros((16,), jnp.float32)

    def body(j, carry):
        for kk in range(width // 16):
            ref2d[j, pl.ds(kk * 16, 16)] = zero
        return carry

    lax.fori_loop(0, nrows, body, 0)


def _stage_edges(ei_hbm, pad_src_hbm, pad_dst_hbm, src_v, dst_v,
                 base, n_real, n_pad):
    pltpu.sync_copy(ei_hbm.at[0, pl.ds(base, n_real)],
                    src_v.at[pl.ds(0, n_real)])
    pltpu.sync_copy(ei_hbm.at[1, pl.ds(base, n_real)],
                    dst_v.at[pl.ds(0, n_real)])
    pltpu.sync_copy(pad_src_hbm.at[pl.ds(0, n_pad)],
                    src_v.at[pl.ds(n_real, n_pad)])
    pltpu.sync_copy(pad_dst_hbm.at[pl.ds(0, n_pad)],
                    dst_v.at[pl.ds(n_real, n_pad)])


def _make_deg_kernel(n_acc, ept, cpt, dw):
    rows_per_tile = n_acc // NS
    n_pad = cpt * CH - ept

    @functools.partial(
        pl.kernel,
        out_type=jax.ShapeDtypeStruct((NC * n_acc, dw), jnp.float32),
        mesh=_sc_mesh(),
        compiler_params=_SC_PARAMS,
        scratch_types=[
            pltpu.VMEM((cpt * CH,), jnp.int32),
            pltpu.VMEM((CH, dw), jnp.float32),
            pltpu.VMEM_SHARED((n_acc, dw), jnp.float32),
            pltpu.SemaphoreType.DMA,
        ],
    )
    def k(ei_hbm, pad_dst_hbm, ones_hbm, zin_hbm, out_hbm, dst_v, ones_v,
          acc_sh, sem):
        c = lax.axis_index("c")
        s = lax.axis_index("s")
        w = c * NS + s
        pltpu.sync_copy(ei_hbm.at[1, pl.ds(w * ept, ept)],
                        dst_v.at[pl.ds(0, ept)])
        pltpu.sync_copy(pad_dst_hbm.at[pl.ds(0, n_pad)],
                        dst_v.at[pl.ds(ept, n_pad)])
        pltpu.sync_copy(ones_hbm, ones_v)
        pltpu.sync_copy(
            zin_hbm.at[pl.ds(s * rows_per_tile, rows_per_tile)],
            acc_sh.at[pl.ds(s * rows_per_tile, rows_per_tile)],
        )
        plsc.subcore_barrier()

        def fire(j, carry):
            pltpu.async_copy(
                ones_v, acc_sh.at[dst_v.at[pl.ds(j * CH, CH)]], sem, add=True
            )
            return carry

        lax.fori_loop(0, cpt, fire, 0)

        def drain(j, carry):
            pltpu.make_async_copy(
                ones_v, acc_sh.at[dst_v.at[pl.ds(j * CH, CH)]], sem
            ).wait()
            return carry

        lax.fori_loop(0, cpt, drain, 0)
        plsc.subcore_barrier()
        pltpu.sync_copy(
            acc_sh.at[pl.ds(s * rows_per_tile, rows_per_tile)],
            out_hbm.at[pl.ds(c * n_acc + s * rows_per_tile, rows_per_tile)],
        )

    return k


def _pipelined_edge_loop(y_sh, acc_sh, src_v, dst_v, rows_v, semg, sems,
                         cpt, width):
    nb = cpt // BG
    nbp = nb // 2
    assert cpt % (2 * BG) == 0

    def gslice(h, i):
        return rows_v.at[h, pl.ds(i * CH, CH)]

    def eslice(v, b, i):
        return v.at[pl.ds((b * BG + i) * CH, CH)]

    def issue_gathers(b, h):
        for i in range(BG):
            pltpu.async_copy(y_sh.at[eslice(src_v, b, i)], gslice(h, i),
                             semg[h])

    def wait_gathers(b, h):
        for i in range(BG):
            pltpu.make_async_copy(y_sh.at[eslice(src_v, b, i)], gslice(h, i),
                                  semg[h]).wait()

    def issue_scatters(b, h):
        for i in range(BG):
            pltpu.async_copy(gslice(h, i), acc_sh.at[eslice(dst_v, b, i)],
                             sems[h], add=True)

    def wait_scatters(b, h):
        for i in range(BG):
            pltpu.make_async_copy(gslice(h, i), acc_sh.at[eslice(dst_v, b, i)],
                                  sems[h]).wait()

    issue_gathers(0, 0)

    def body(bp, carry):
        b0 = 2 * bp
        wait_gathers(b0, 0)
        issue_scatters(b0, 0)

        @pl.when(bp > 0)
        def _():
            wait_scatters(b0 - 1, 1)

        issue_gathers(b0 + 1, 1)
        wait_gathers(b0 + 1, 1)
        issue_scatters(b0 + 1, 1)

        @pl.when(bp < nbp - 1)
        def _():
            wait_scatters(b0, 0)
            issue_gathers(b0 + 2, 0)

        return carry

    lax.fori_loop(0, nbp, body, 0)
    wait_scatters(nb - 2, 0)
    wait_scatters(nb - 1, 1)


def _rsqrt_newton(x):
    i = plsc.bitcast(x, jnp.int32)
    i = jnp.full((16,), 0x5F3759DF, jnp.int32) - lax.shift_right_logical(
        i, jnp.full((16,), 1, jnp.int32)
    )
    y = plsc.bitcast(i, jnp.float32)
    for _ in range(3):
        y = y * (1.5 - 0.5 * x * y * y)
    return y


def _make_l1_kernel(n_acc, ept, cpt, width, dw):
    rows_per_tile = n_acc // NS
    colw = width // 2
    n_pad = cpt * CH - ept

    @functools.partial(
        pl.kernel,
        out_type=(
            jax.ShapeDtypeStruct((n_acc, width), jnp.float32),
            jax.ShapeDtypeStruct((n_acc,), jnp.float32),
        ),
        mesh=_sc_mesh(),
        compiler_params=_SC_PARAMS,
        scratch_types=[
            pltpu.VMEM((cpt * CH,), jnp.int32),
            pltpu.VMEM((cpt * CH,), jnp.int32),
            pltpu.VMEM((2, BG * CH, colw), jnp.float32),
            pltpu.VMEM((rows_per_tile, dw), jnp.float32),
            pltpu.VMEM((rows_per_tile, dw), jnp.float32),
            pltpu.VMEM((rows_per_tile,), jnp.float32),
            pltpu.VMEM_SHARED((n_acc, colw), jnp.float32),
            pltpu.VMEM_SHARED((n_acc, colw), jnp.float32),
            pltpu.SemaphoreType.DMA,
            pltpu.SemaphoreType.DMA,
            pltpu.SemaphoreType.DMA,
            pltpu.SemaphoreType.DMA,
        ],
    )
    def k(xw_hbm, degacc_hbm, ei_hbm, pad_src_hbm, pad_dst_hbm, zin_hbm,
          out_hbm, dinv_hbm, src_v, dst_v, rows_v, dbuf0, dbuf1, dinv_v,
          acc_sh, y_sh, semg0, semg1, sems0, sems1):
        c = lax.axis_index("c")
        s = lax.axis_index("s")
        xwb = rows_v.at[0, pl.ds(0, rows_per_tile)]
        _stage_edges(ei_hbm, pad_src_hbm, pad_dst_hbm,
                     src_v, dst_v, s * ept, ept, n_pad)
        pltpu.sync_copy(
            degacc_hbm.at[pl.ds(s * rows_per_tile, rows_per_tile)], dbuf0)
        pltpu.sync_copy(
            degacc_hbm.at[pl.ds(n_acc + s * rows_per_tile, rows_per_tile)],
            dbuf1)
        pltpu.sync_copy(
            xw_hbm.at[pl.ds(s * rows_per_tile, rows_per_tile),
                      pl.ds(c * colw, colw)],
            xwb,
        )
        lanes = lax.iota(jnp.int32, 16)
        zcol = jnp.zeros((16,), jnp.int32)

        def dv(j, carry):
            rows_i = j * 16 + lanes
            d0 = plsc.load_gather(dbuf0, [rows_i, zcol])
            d1 = plsc.load_gather(dbuf1, [rows_i, zcol])
            dinv_v[pl.ds(j * 16, 16)] = _rsqrt_newton(d0 + d1 + 1.0)
            return carry

        lax.fori_loop(0, rows_per_tile // 16, dv, 0)
        pltpu.sync_copy(dinv_v, dinv_hbm.at[pl.ds(s * rows_per_tile,
                                                  rows_per_tile)])

        def scale_rows(j, carry):
            dvec = dinv_v[pl.ds(j * 16, 16)]
            for r in range(16):
                row = j * 16 + r
                d = dvec[r]
                for kk in range(colw // 16):
                    rows_v[0, row, pl.ds(kk * 16, 16)] = (
                        rows_v[0, row, pl.ds(kk * 16, 16)] * d
                    )
            return carry

        lax.fori_loop(0, rows_per_tile // 16, scale_rows, 0)
        pltpu.sync_copy(xwb, y_sh.at[pl.ds(s * rows_per_tile,
                                           rows_per_tile)])
        pltpu.sync_copy(
            zin_hbm.at[pl.ds(s * rows_per_tile, rows_per_tile)],
            acc_sh.at[pl.ds(s * rows_per_tile, rows_per_tile)],
        )
        plsc.subcore_barrier()
        _pipelined_edge_loop(y_sh, acc_sh, src_v, dst_v, rows_v,
                             (semg0, semg1), (sems0, sems1), cpt, colw)
        plsc.subcore_barrier()
        pltpu.sync_copy(
            acc_sh.at[pl.ds(s * rows_per_tile, rows_per_tile)],
            out_hbm.at[pl.ds(s * rows_per_tile, rows_per_tile),
                       pl.ds(c * colw, colw)],
        )

    return k


def _make_colsplit_kernel(n_acc, ept, cpt, width):
    rows_per_tile = n_acc // NS
    colw = width // 2
    n_pad = cpt * CH - ept

    @functools.partial(
        pl.kernel,
        out_type=jax.ShapeDtypeStruct((n_acc, width), jnp.float32),
        mesh=_sc_mesh(),
        compiler_params=_SC_PARAMS,
        scratch_types=[
            pltpu.VMEM((cpt * CH,), jnp.int32),
            pltpu.VMEM((cpt * CH,), jnp.int32),
            pltpu.VMEM((2, BG * CH, colw), jnp.float32),
            pltpu.VMEM_SHARED((n_acc, colw), jnp.float32),
            pltpu.VMEM_SHARED((n_acc, colw), jnp.float32),
            pltpu.SemaphoreType.DMA,
            pltpu.SemaphoreType.DMA,
            pltpu.SemaphoreType.DMA,
            pltpu.SemaphoreType.DMA,
        ],
    )
    def k(y_hbm, ei_hbm, pad_src_hbm, pad_dst_hbm, zin_hbm, out_hbm,
          src_v, dst_v, rows_v, acc_sh, y_sh, semg0, semg1, sems0, sems1):
        c = lax.axis_index("c")
        s = lax.axis_index("s")
        _stage_edges(ei_hbm, pad_src_hbm, pad_dst_hbm,
                     src_v, dst_v, s * ept, ept, n_pad)
        pltpu.sync_copy(
            y_hbm.at[pl.ds(s * rows_per_tile, rows_per_tile),
                     pl.ds(c * colw, colw)],
            y_sh.at[pl.ds(s * rows_per_tile, rows_per_tile)],
        )
        pltpu.sync_copy(
            zin_hbm.at[pl.ds(s * rows_per_tile, rows_per_tile)],
            acc_sh.at[pl.ds(s * rows_per_tile, rows_per_tile)],
        )
        plsc.subcore_barrier()
        _pipelined_edge_loop(y_sh, acc_sh, src_v, dst_v, rows_v,
                             (semg0, semg1), (sems0, sems1), cpt, colw)
        plsc.subcore_barrier()
        pltpu.sync_copy(
            acc_sh.at[pl.ds(s * rows_per_tile, rows_per_tile)],
            out_hbm.at[pl.ds(s * rows_per_tile, rows_per_tile),
                       pl.ds(c * colw, colw)],
        )

    return k


def _make_scatter_kernel(n_acc, ept, cpt, width):
    rows_per_tile = n_acc // NS
    n_pad = cpt * CH - ept

    @functools.partial(
        pl.kernel,
        out_type=jax.ShapeDtypeStruct((NC, n_acc, width), jnp.float32),
        mesh=_sc_mesh(),
        compiler_params=_SC_PARAMS,
        scratch_types=[
            pltpu.VMEM((cpt * CH,), jnp.int32),
            pltpu.VMEM((cpt * CH,), jnp.int32),
            pltpu.VMEM((2, BG * CH, width), jnp.float32),
            pltpu.VMEM_SHARED((n_acc, width), jnp.float32),
            pltpu.VMEM_SHARED((n_acc, width), jnp.float32),
            pltpu.SemaphoreType.DMA,
            pltpu.SemaphoreType.DMA,
            pltpu.SemaphoreType.DMA,
            pltpu.SemaphoreType.DMA,
        ],
    )
    def k(y_hbm, ei_hbm, pad_src_hbm, pad_dst_hbm, zin_hbm, out_hbm,
          src_v, dst_v, rows_v, acc_sh, y_sh, semg0, semg1, sems0, sems1):
        c = lax.axis_index("c")
        s = lax.axis_index("s")
        w = c * NS + s
        _stage_edges(ei_hbm, pad_src_hbm, pad_dst_hbm,
                     src_v, dst_v, w * ept, ept, n_pad)
        pltpu.sync_copy(
            y_hbm.at[pl.ds(s * rows_per_tile, rows_per_tile)],
            y_sh.at[pl.ds(s * rows_per_tile, rows_per_tile)],
        )
        pltpu.sync_copy(
            zin_hbm.at[pl.ds(s * rows_per_tile, rows_per_tile)],
            acc_sh.at[pl.ds(s * rows_per_tile, rows_per_tile)],
        )
        plsc.subcore_barrier()
        _pipelined_edge_loop(y_sh, acc_sh, src_v, dst_v, rows_v,
                             (semg0, semg1), (sems0, sems1), cpt, width)
        plsc.subcore_barrier()
        pltpu.sync_copy(
            acc_sh.at[pl.ds(s * rows_per_tile, rows_per_tile)],
            out_hbm.at[c, pl.ds(s * rows_per_tile, rows_per_tile)],
        )

    return k


def _tc1a_body(x_ref, w1_ref, xw_ref):
    xw_ref[...] = jnp.dot(
        x_ref[...], w1_ref[...], preferred_element_type=jnp.float32
    )


def _tc2_body(s1_ref, xw_ref, dinv_ref, w2_ref, b1_ref, y2_ref):
    dinv = dinv_ref[...][:, 0:1]
    y1 = xw_ref[...] * dinv
    h = jnp.maximum((s1_ref[...] + y1) * dinv + b1_ref[...], 0.0)
    hw = jnp.dot(h, w2_ref[...], preferred_element_type=jnp.float32)
    y2_ref[...] = hw * dinv


def _tc3_body(s2_ref, y2_ref, dinv_ref, b2_ref, out_ref):
    s2 = s2_ref[...]
    dinv = dinv_ref[...][:, 0:1]
    z = (s2[0] + s2[1] + y2_ref[...]) * dinv + b2_ref[...]
    m = jnp.max(z, axis=1, keepdims=True)
    e = jnp.exp(z - m)
    out_ref[...] = z - m - jnp.log(jnp.sum(e, axis=1, keepdims=True))


def kernel(x, edge_index, W1, b1, W2, b2):
    n, d_in = x.shape
    e = edge_index.shape[1]
    h_dim = W1.shape[1]
    c_dim = W2.shape[1]
    assert e % NW == 0

    n_acc = (n // 512 + 1) * 512
    n_dummy = n_acc - n

    ept32 = e // NW
    cpt32 = -(-ept32 // (CH * 2 * BG)) * 2 * BG
    ept16 = e // NS
    cpt16 = -(-ept16 // (CH * 2 * BG)) * 2 * BG
    max_pad = max(cpt32 * CH - ept32, cpt16 * CH - ept16)
    pad_src = jnp.zeros((max_pad,), jnp.int32)
    pad_dst = n + jnp.arange(max_pad, dtype=jnp.int32) % n_dummy

    x_pad = jnp.concatenate([x, jnp.zeros((n_acc - n, d_in), x.dtype)])

    dw = 8

    degacc = _make_deg_kernel(n_acc, ept32, cpt32, dw)(
        edge_index, pad_dst, jnp.ones((CH, dw), jnp.float32),
        jnp.zeros((n_acc, dw), jnp.float32),
    )
    xw = pl.pallas_call(
        _tc1a_body,
        out_shape=jax.ShapeDtypeStruct((n_acc, h_dim), jnp.float32),
    )(x_pad, W1)

    zin_h2 = jnp.zeros((n_acc, h_dim // 2), jnp.float32)
    s1, dinv_flat = _make_l1_kernel(n_acc, ept16, cpt16, h_dim, dw)(
        xw, degacc, edge_index, pad_src, pad_dst, zin_h2
    )
    dinv = dinv_flat.reshape(n_acc, 1)

    y2 = pl.pallas_call(
        _tc2_body,
        out_shape=jax.ShapeDtypeStruct((n_acc, c_dim), jnp.float32),
    )(s1, xw, dinv, W2, b1.reshape(1, h_dim))

    zin_c = jnp.zeros((n_acc, c_dim), jnp.float32)
    s2 = _make_scatter_kernel(n_acc, ept32, cpt32, c_dim)(
        y2, edge_index, pad_src, pad_dst, zin_c
    )

    out = pl.pallas_call(
        _tc3_body,
        grid=(1,),
        in_specs=[
            pl.BlockSpec((NC, n, c_dim), lambda i: (0, 0, 0)),
            pl.BlockSpec((n, c_dim), lambda i: (0, 0)),
            pl.BlockSpec((n, 1), lambda i: (0, 0)),
            pl.BlockSpec((1, c_dim), lambda i: (0, 0)),
        ],
        out_specs=pl.BlockSpec((n, c_dim), lambda i: (0, 0)),
        out_shape=jax.ShapeDtypeStruct((n, c_dim), jnp.float32),
    )(s2, y2, dinv, b2.reshape(1, c_dim))

    return out


# speedup vs baseline: 1.2571x; 1.0036x over previous
import functools

import jax
import jax.numpy as jnp
from jax import lax
from jax.experimental import pallas as pl
from jax.experimental.pallas import tpu as pltpu
from jax.experimental.pallas import tpu_sc as plsc

NC = 2
NS = 16
NW = NC * NS

CH = 128
BG = 4


def _sc_mesh():
    return plsc.VectorSubcoreMesh(
        core_axis_name="c", subcore_axis_name="s", num_cores=NC, num_subcores=NS
    )


_SC_PARAMS = pltpu.CompilerParams(
    use_tc_tiling_on_sc=False, needs_layout_passes=False
)


def _zero_fill(ref2d, nrows, width):
    zero = jnp.zeros((16,), jnp.float32)

    def body(j, carry):
        for kk in range(width // 16):
            ref2d[j, pl.ds(kk * 16, 16)] = zero
        return carry

    lax.fori_loop(0, nrows, body, 0)


def _stage_edges(ei_hbm, pad_src_hbm, pad_dst_hbm, src_v, dst_v,
                 base, n_real, n_pad):
    pltpu.sync_copy(ei_hbm.at[0, pl.ds(base, n_real)],
                    src_v.at[pl.ds(0, n_real)])
    pltpu.sync_copy(ei_hbm.at[1, pl.ds(base, n_real)],
                    dst_v.at[pl.ds(0, n_real)])
    pltpu.sync_copy(pad_src_hbm.at[pl.ds(0, n_pad)],
                    src_v.at[pl.ds(n_real, n_pad)])
    pltpu.sync_copy(pad_dst_hbm.at[pl.ds(0, n_pad)],
                    dst_v.at[pl.ds(n_real, n_pad)])


def _make_deg_kernel(n_acc, ept, cpt, dw):
    rows_per_tile = n_acc // NS
    n_pad = cpt * CH - ept

    @functools.partial(
        pl.kernel,
        out_type=jax.ShapeDtypeStruct((NC * n_acc, dw), jnp.float32),
        mesh=_sc_mesh(),
        compiler_params=_SC_PARAMS,
        scratch_types=[
            pltpu.VMEM((cpt * CH,), jnp.int32),
            pltpu.VMEM((CH, dw), jnp.float32),
            pltpu.VMEM_SHARED((n_acc, dw), jnp.float32),
            pltpu.SemaphoreType.DMA,
        ],
    )
    def k(ei_hbm, pad_dst_hbm, ones_hbm, zin_hbm, out_hbm, dst_v, ones_v,
          acc_sh, sem):
        c = lax.axis_index("c")
        s = lax.axis_index("s")
        w = c * NS + s
        pltpu.sync_copy(ei_hbm.at[1, pl.ds(w * ept, ept)],
                        dst_v.at[pl.ds(0, ept)])
        pltpu.sync_copy(pad_dst_hbm.at[pl.ds(0, n_pad)],
                        dst_v.at[pl.ds(ept, n_pad)])
        pltpu.sync_copy(ones_hbm, ones_v)
        pltpu.sync_copy(
            zin_hbm.at[pl.ds(s * rows_per_tile, rows_per_tile)],
            acc_sh.at[pl.ds(s * rows_per_tile, rows_per_tile)],
        )
        plsc.subcore_barrier()

        def fire(j, carry):
            pltpu.async_copy(
                ones_v, acc_sh.at[dst_v.at[pl.ds(j * CH, CH)]], sem, add=True
            )
            return carry

        lax.fori_loop(0, cpt, fire, 0)

        def drain(j, carry):
            pltpu.make_async_copy(
                ones_v, acc_sh.at[dst_v.at[pl.ds(j * CH, CH)]], sem
            ).wait()
            return carry

        lax.fori_loop(0, cpt, drain, 0)
        plsc.subcore_barrier()
        pltpu.sync_copy(
            acc_sh.at[pl.ds(s * rows_per_tile, rows_per_tile)],
            out_hbm.at[pl.ds(c * n_acc + s * rows_per_tile, rows_per_tile)],
        )

    return k


def _pipelined_edge_loop(y_sh, acc_sh, src_v, dst_v, rows_v, semg, sems,
                         cpt, width, bg=BG):
    nb = cpt // bg
    nbp = nb // 2
    assert cpt % (2 * bg) == 0

    def gslice(h, i):
        return rows_v.at[h, pl.ds(i * CH, CH)]

    def eslice(v, b, i):
        return v.at[pl.ds((b * bg + i) * CH, CH)]

    def issue_gathers(b, h):
        for i in range(bg):
            pltpu.async_copy(y_sh.at[eslice(src_v, b, i)], gslice(h, i),
                             semg[h])

    def wait_gathers(b, h):
        for i in range(bg):
            pltpu.make_async_copy(y_sh.at[eslice(src_v, b, i)], gslice(h, i),
                                  semg[h]).wait()

    def issue_scatters(b, h):
        for i in range(bg):
            pltpu.async_copy(gslice(h, i), acc_sh.at[eslice(dst_v, b, i)],
                             sems[h], add=True)

    def wait_scatters(b, h):
        for i in range(bg):
            pltpu.make_async_copy(gslice(h, i), acc_sh.at[eslice(dst_v, b, i)],
                                  sems[h]).wait()

    issue_gathers(0, 0)

    def body(bp, carry):
        b0 = 2 * bp
        wait_gathers(b0, 0)
        issue_scatters(b0, 0)

        @pl.when(bp > 0)
        def _():
            wait_scatters(b0 - 1, 1)

        issue_gathers(b0 + 1, 1)
        wait_gathers(b0 + 1, 1)
        issue_scatters(b0 + 1, 1)

        @pl.when(bp < nbp - 1)
        def _():
            wait_scatters(b0, 0)
            issue_gathers(b0 + 2, 0)

        return carry

    lax.fori_loop(0, nbp, body, 0)
    wait_scatters(nb - 2, 0)
    wait_scatters(nb - 1, 1)


def _rsqrt_newton(x):
    i = plsc.bitcast(x, jnp.int32)
    i = jnp.full((16,), 0x5F3759DF, jnp.int32) - lax.shift_right_logical(
        i, jnp.full((16,), 1, jnp.int32)
    )
    y = plsc.bitcast(i, jnp.float32)
    for _ in range(3):
        y = y * (1.5 - 0.5 * x * y * y)
    return y


def _make_l1_kernel(n_acc, ept, cpt, width, dw):
    rows_per_tile = n_acc // NS
    colw = width // 2
    n_pad = cpt * CH - ept

    @functools.partial(
        pl.kernel,
        out_type=(
            jax.ShapeDtypeStruct((n_acc, width), jnp.float32),
            jax.ShapeDtypeStruct((n_acc,), jnp.float32),
        ),
        mesh=_sc_mesh(),
        compiler_params=_SC_PARAMS,
        scratch_types=[
            pltpu.VMEM((cpt * CH,), jnp.int32),
            pltpu.VMEM((cpt * CH,), jnp.int32),
            pltpu.VMEM((2, BG * CH, colw), jnp.float32),
            pltpu.VMEM((rows_per_tile, dw), jnp.float32),
            pltpu.VMEM((rows_per_tile, dw), jnp.float32),
            pltpu.VMEM((rows_per_tile,), jnp.float32),
            pltpu.VMEM_SHARED((n_acc, colw), jnp.float32),
            pltpu.VMEM_SHARED((n_acc, colw), jnp.float32),
            pltpu.SemaphoreType.DMA,
            pltpu.SemaphoreType.DMA,
            pltpu.SemaphoreType.DMA,
            pltpu.SemaphoreType.DMA,
        ],
    )
    def k(xw_hbm, degacc_hbm, ei_hbm, pad_src_hbm, pad_dst_hbm, zin_hbm,
          out_hbm, dinv_hbm, src_v, dst_v, rows_v, dbuf0, dbuf1, dinv_v,
          acc_sh, y_sh, semg0, semg1, sems0, sems1):
        c = lax.axis_index("c")
        s = lax.axis_index("s")
        xwb = rows_v.at[0, pl.ds(0, rows_per_tile)]
        _stage_edges(ei_hbm, pad_src_hbm, pad_dst_hbm,
                     src_v, dst_v, s * ept, ept, n_pad)
        pltpu.sync_copy(
            degacc_hbm.at[pl.ds(s * rows_per_tile, rows_per_tile)], dbuf0)
        pltpu.sync_copy(
            degacc_hbm.at[pl.ds(n_acc + s * rows_per_tile, rows_per_tile)],
            dbuf1)
        pltpu.sync_copy(
            xw_hbm.at[pl.ds(s * rows_per_tile, rows_per_tile),
                      pl.ds(c * colw, colw)],
            xwb,
        )
        lanes = lax.iota(jnp.int32, 16)
        zcol = jnp.zeros((16,), jnp.int32)

        def dv(j, carry):
            rows_i = j * 16 + lanes
            d0 = plsc.load_gather(dbuf0, [rows_i, zcol])
            d1 = plsc.load_gather(dbuf1, [rows_i, zcol])
            dinv_v[pl.ds(j * 16, 16)] = _rsqrt_newton(d0 + d1 + 1.0)
            return carry

        lax.fori_loop(0, rows_per_tile // 16, dv, 0)
        pltpu.sync_copy(dinv_v, dinv_hbm.at[pl.ds(s * rows_per_tile,
                                                  rows_per_tile)])

        def scale_rows(j, carry):
            dvec = dinv_v[pl.ds(j * 16, 16)]
            for r in range(16):
                row = j * 16 + r
                d = dvec[r]
                for kk in range(colw // 16):
                    rows_v[0, row, pl.ds(kk * 16, 16)] = (
                        rows_v[0, row, pl.ds(kk * 16, 16)] * d
                    )
            return carry

        lax.fori_loop(0, rows_per_tile // 16, scale_rows, 0)
        pltpu.sync_copy(xwb, y_sh.at[pl.ds(s * rows_per_tile,
                                           rows_per_tile)])
        pltpu.sync_copy(
            zin_hbm.at[pl.ds(s * rows_per_tile, rows_per_tile)],
            acc_sh.at[pl.ds(s * rows_per_tile, rows_per_tile)],
        )
        plsc.subcore_barrier()
        _pipelined_edge_loop(y_sh, acc_sh, src_v, dst_v, rows_v,
                             (semg0, semg1), (sems0, sems1), cpt, colw)
        plsc.subcore_barrier()
        pltpu.sync_copy(
            acc_sh.at[pl.ds(s * rows_per_tile, rows_per_tile)],
            out_hbm.at[pl.ds(s * rows_per_tile, rows_per_tile),
                       pl.ds(c * colw, colw)],
        )

    return k


def _make_colsplit_kernel(n_acc, ept, cpt, width):
    rows_per_tile = n_acc // NS
    colw = width // 2
    n_pad = cpt * CH - ept

    @functools.partial(
        pl.kernel,
        out_type=jax.ShapeDtypeStruct((n_acc, width), jnp.float32),
        mesh=_sc_mesh(),
        compiler_params=_SC_PARAMS,
        scratch_types=[
            pltpu.VMEM((cpt * CH,), jnp.int32),
            pltpu.VMEM((cpt * CH,), jnp.int32),
            pltpu.VMEM((2, BG * CH, colw), jnp.float32),
            pltpu.VMEM_SHARED((n_acc, colw), jnp.float32),
            pltpu.VMEM_SHARED((n_acc, colw), jnp.float32),
            pltpu.SemaphoreType.DMA,
            pltpu.SemaphoreType.DMA,
            pltpu.SemaphoreType.DMA,
            pltpu.SemaphoreType.DMA,
        ],
    )
    def k(y_hbm, ei_hbm, pad_src_hbm, pad_dst_hbm, zin_hbm, out_hbm,
          src_v, dst_v, rows_v, acc_sh, y_sh, semg0, semg1, sems0, sems1):
        c = lax.axis_index("c")
        s = lax.axis_index("s")
        _stage_edges(ei_hbm, pad_src_hbm, pad_dst_hbm,
                     src_v, dst_v, s * ept, ept, n_pad)
        pltpu.sync_copy(
            y_hbm.at[pl.ds(s * rows_per_tile, rows_per_tile),
                     pl.ds(c * colw, colw)],
            y_sh.at[pl.ds(s * rows_per_tile, rows_per_tile)],
        )
        pltpu.sync_copy(
            zin_hbm.at[pl.ds(s * rows_per_tile, rows_per_tile)],
            acc_sh.at[pl.ds(s * rows_per_tile, rows_per_tile)],
        )
        plsc.subcore_barrier()
        _pipelined_edge_loop(y_sh, acc_sh, src_v, dst_v, rows_v,
                             (semg0, semg1), (sems0, sems1), cpt, colw)
        plsc.subcore_barrier()
        pltpu.sync_copy(
            acc_sh.at[pl.ds(s * rows_per_tile, rows_per_tile)],
            out_hbm.at[pl.ds(s * rows_per_tile, rows_per_tile),
                       pl.ds(c * colw, colw)],
        )

    return k


def _make_scatter_kernel(n_acc, ept, cpt, width):
    rows_per_tile = n_acc // NS
    n_pad = cpt * CH - ept

    @functools.partial(
        pl.kernel,
        out_type=jax.ShapeDtypeStruct((NC, n_acc, width), jnp.float32),
        mesh=_sc_mesh(),
        compiler_params=_SC_PARAMS,
        scratch_types=[
            pltpu.VMEM((cpt * CH,), jnp.int32),
            pltpu.VMEM((cpt * CH,), jnp.int32),
            pltpu.VMEM((2, 8 * CH, width), jnp.float32),
            pltpu.VMEM_SHARED((n_acc, width), jnp.float32),
            pltpu.VMEM_SHARED((n_acc, width), jnp.float32),
            pltpu.SemaphoreType.DMA,
            pltpu.SemaphoreType.DMA,
            pltpu.SemaphoreType.DMA,
            pltpu.SemaphoreType.DMA,
        ],
    )
    def k(y_hbm, ei_hbm, pad_src_hbm, pad_dst_hbm, zin_hbm, out_hbm,
          src_v, dst_v, rows_v, acc_sh, y_sh, semg0, semg1, sems0, sems1):
        c = lax.axis_index("c")
        s = lax.axis_index("s")
        w = c * NS + s
        _stage_edges(ei_hbm, pad_src_hbm, pad_dst_hbm,
                     src_v, dst_v, w * ept, ept, n_pad)
        pltpu.sync_copy(
            y_hbm.at[pl.ds(s * rows_per_tile, rows_per_tile)],
            y_sh.at[pl.ds(s * rows_per_tile, rows_per_tile)],
        )
        pltpu.sync_copy(
            zin_hbm.at[pl.ds(s * rows_per_tile, rows_per_tile)],
            acc_sh.at[pl.ds(s * rows_per_tile, rows_per_tile)],
        )
        plsc.subcore_barrier()
        _pipelined_edge_loop(y_sh, acc_sh, src_v, dst_v, rows_v,
                             (semg0, semg1), (sems0, sems1), cpt, width, bg=8)
        plsc.subcore_barrier()
        pltpu.sync_copy(
            acc_sh.at[pl.ds(s * rows_per_tile, rows_per_tile)],
            out_hbm.at[c, pl.ds(s * rows_per_tile, rows_per_tile)],
        )

    return k


def _tc1a_body(x_ref, w1_ref, xw_ref):
    xw_ref[...] = jnp.dot(
        x_ref[...], w1_ref[...], preferred_element_type=jnp.float32
    )


def _tc2_body(s1_ref, xw_ref, dinv_ref, w2_ref, b1_ref, y2_ref):
    dinv = dinv_ref[...][:, 0:1]
    y1 = xw_ref[...] * dinv
    h = jnp.maximum((s1_ref[...] + y1) * dinv + b1_ref[...], 0.0)
    hw = jnp.dot(h, w2_ref[...], preferred_element_type=jnp.float32)
    y2_ref[...] = hw * dinv


def _tc3_body(s2_ref, y2_ref, dinv_ref, b2_ref, out_ref):
    s2 = s2_ref[...]
    dinv = dinv_ref[...][:, 0:1]
    z = (s2[0] + s2[1] + y2_ref[...]) * dinv + b2_ref[...]
    m = jnp.max(z, axis=1, keepdims=True)
    e = jnp.exp(z - m)
    out_ref[...] = z - m - jnp.log(jnp.sum(e, axis=1, keepdims=True))


def kernel(x, edge_index, W1, b1, W2, b2):
    n, d_in = x.shape
    e = edge_index.shape[1]
    h_dim = W1.shape[1]
    c_dim = W2.shape[1]
    assert e % NW == 0

    n_acc = (n // 512 + 1) * 512
    n_dummy = n_acc - n

    ept32 = e // NW
    cpt32 = -(-ept32 // (CH * 2 * BG)) * 2 * BG
    ept16 = e // NS
    cpt16 = -(-ept16 // (CH * 2 * BG)) * 2 * BG
    max_pad = max(cpt32 * CH - ept32, cpt16 * CH - ept16)
    pad_src = jnp.zeros((max_pad,), jnp.int32)
    pad_dst = n + jnp.arange(max_pad, dtype=jnp.int32) % n_dummy

    x_pad = jnp.concatenate([x, jnp.zeros((n_acc - n, d_in), x.dtype)])

    dw = 8

    degacc = _make_deg_kernel(n_acc, ept32, cpt32, dw)(
        edge_index, pad_dst, jnp.ones((CH, dw), jnp.float32),
        jnp.zeros((n_acc, dw), jnp.float32),
    )
    xw = pl.pallas_call(
        _tc1a_body,
        out_shape=jax.ShapeDtypeStruct((n_acc, h_dim), jnp.float32),
    )(x_pad, W1)

    zin_h2 = jnp.zeros((n_acc, h_dim // 2), jnp.float32)
    s1, dinv_flat = _make_l1_kernel(n_acc, ept16, cpt16, h_dim, dw)(
        xw, degacc, edge_index, pad_src, pad_dst, zin_h2
    )
    dinv = dinv_flat.reshape(n_acc, 1)

    y2 = pl.pallas_call(
        _tc2_body,
        out_shape=jax.ShapeDtypeStruct((n_acc, c_dim), jnp.float32),
    )(s1, xw, dinv, W2, b1.reshape(1, h_dim))

    zin_c = jnp.zeros((n_acc, c_dim), jnp.float32)
    s2 = _make_scatter_kernel(n_acc, ept32, cpt32, c_dim)(
        y2, edge_index, pad_src, pad_dst, zin_c
    )

    out = pl.pallas_call(
        _tc3_body,
        grid=(1,),
        in_specs=[
            pl.BlockSpec((NC, n, c_dim), lambda i: (0, 0, 0)),
            pl.BlockSpec((n, c_dim), lambda i: (0, 0)),
            pl.BlockSpec((n, 1), lambda i: (0, 0)),
            pl.BlockSpec((1, c_dim), lambda i: (0, 0)),
        ],
        out_specs=pl.BlockSpec((n, c_dim), lambda i: (0, 0)),
        out_shape=jax.ShapeDtypeStruct((n, c_dim), jnp.float32),
    )(s2, y2, dinv, b2.reshape(1, c_dim))

    return out
